# baseline (speedup 1.0000x reference)
import functools

import jax
import jax.numpy as jnp
from jax import lax
from jax.experimental import pallas as pl
from jax.experimental.pallas import tpu as pltpu
from jax.experimental.pallas import tpu_sc as plsc

_C = 81
_G = 400
_GP = 512
_BINS = 416
_PAD_BIN = 415
_TBL = 408

_NW = 32
_CHUNK = 1568
_NP = _NW * _CHUNK

_BA = 896


def _match_body(bbt_ref, gtp_ref, out_ref, *, n):
    b = bbt_ref[...]
    g = gtp_ref[...]
    ax1, ay1, ax2, ay2 = (b[i:i + 1, :] for i in range(4))
    gx1, gy1, gx2, gy2 = (g[:, i:i + 1] for i in range(4))

    area_a = (ax2 - ax1) * (ay2 - ay1)
    area_g = (gx2 - gx1) * (gy2 - gy1)
    w = jnp.maximum(jnp.minimum(ax2, gx2) - jnp.maximum(ax1, gx1), 0.0)
    h = jnp.maximum(jnp.minimum(ay2, gy2) - jnp.maximum(ay1, gy1), 0.0)
    inter = w * h
    r = inter / (area_a + area_g)

    bits = lax.bitcast_convert_type(r, jnp.int32)
    revj = (_GP - 1) - lax.broadcasted_iota(jnp.int32, (_GP, 1), 0)
    key = jnp.bitwise_or(jnp.bitwise_and(bits, -_GP), revj)
    kmax = jnp.max(key, axis=0, keepdims=True)

    idx = (_GP - 1) - jnp.bitwise_and(kmax, _GP - 1)
    r_q = lax.bitcast_convert_type(jnp.bitwise_and(kmax, -_GP), jnp.float32)
    matched = jnp.where(r_q >= jnp.float32(1.0 / 3.0), idx, _G)
    aid = pl.program_id(0) * _BA + lax.broadcasted_iota(jnp.int32, (1, _BA), 1)
    out_ref[...] = jnp.where(aid < n, matched, _PAD_BIN)


def _match(bbt, gtp, n):
    return pl.pallas_call(
        functools.partial(_match_body, n=n),
        grid=(_NP // _BA,),
        in_specs=[
            pl.BlockSpec((4, _BA), lambda i: (0, i)),
            pl.BlockSpec((_GP, 4), lambda i: (0, 0)),
        ],
        out_specs=pl.BlockSpec((1, _BA), lambda i: (0, i)),
        out_shape=jax.ShapeDtypeStruct((1, _NP), jnp.int32),
    )(bbt, gtp)


def _hist_body(idx_hbm, out_hbm, idx_v, hist_v, row_v):
    wid = lax.axis_index("s") * 2 + lax.axis_index("c")
    base = wid * _CHUNK
    pltpu.sync_copy(idx_hbm.at[pl.ds(base, _CHUNK)], idx_v)

    zeros = jnp.zeros((16,), jnp.float32)
    ones = jnp.ones((16,), jnp.float32)
    lane_off = lax.iota(jnp.int32, 16) * _BINS

    def _zero(i, _):
        for k in range(16):
            hist_v[pl.ds((i * 16 + k) * 16, 16)] = zeros
        return 0
    lax.fori_loop(0, _BINS // 16, _zero, 0)

    def _accum(i, _):
        for k in range(7):
            v = idx_v[pl.ds((i * 7 + k) * 16, 16)]
            plsc.addupdate_scatter(hist_v, [lane_off + v], ones)
        return 0
    lax.fori_loop(0, _CHUNK // 16 // 7, _accum, 0)

    def _reduce(c, _):
        acc = hist_v[pl.ds(c * 16, 16)]
        for r in range(1, 16):
            acc = acc + hist_v[pl.ds(c * 16 + r * _BINS, 16)]
        row_v[pl.ds(c * 16, 16)] = acc
        return 0
    lax.fori_loop(0, _BINS // 16, _reduce, 0)

    pltpu.sync_copy(row_v, out_hbm.at[wid])


def _hist_sc(flat_idx):
    return pl.kernel(
        _hist_body,
        out_type=jax.ShapeDtypeStruct((_NW, _BINS), jnp.float32),
        mesh=plsc.VectorSubcoreMesh(
            core_axis_name="c", subcore_axis_name="s",
            num_cores=2, num_subcores=16),
        scratch_types=[
            pltpu.VMEM((_CHUNK,), jnp.int32),
            pltpu.VMEM((16 * _BINS,), jnp.float32),
            pltpu.VMEM((_BINS,), jnp.float32),
        ],
        compiler_params=pltpu.CompilerParams(needs_layout_passes=False),
    )(flat_idx)


def _combine_body(hist_ref, clst_ref, lab_ref, bbt_ref, glt_ref, out_ref, n):
    counts = jnp.sum(hist_ref[...], axis=0, keepdims=True)[:, :_TBL]
    bin_id = lax.broadcasted_iota(jnp.int32, (1, _TBL), 1)
    counts = jnp.where(bin_id <= _G, counts, 0.0)

    logits = clst_ref[...]
    m = jnp.max(logits, axis=0, keepdims=True)
    sh = logits - m
    lse = jnp.log(jnp.sum(jnp.exp(sh), axis=0, keepdims=True))
    row = lax.broadcasted_iota(jnp.int32, (_C, _TBL), 0)
    sel = jnp.sum(jnp.where(row == lab_ref[...], sh, 0.0), axis=0, keepdims=True)
    l_cls = lse - sel

    b = bbt_ref[...]
    g = glt_ref[...]
    d = jnp.abs(b - g)
    sl1 = jnp.where(d < 1.0, 0.5 * d * d, d - 0.5)
    l_reg = jnp.sum(sl1, axis=0, keepdims=True)

    area_b = (b[2:3] - b[0:1]) * (b[3:4] - b[1:2])
    area_g = (g[2:3] - g[0:1]) * (g[3:4] - g[1:2])
    w = jnp.maximum(jnp.minimum(b[2:3], g[2:3]) - jnp.maximum(b[0:1], g[0:1]), 0.0)
    h = jnp.maximum(jnp.minimum(b[3:4], g[3:4]) - jnp.maximum(b[1:2], g[1:2]), 0.0)
    inter = w * h
    union = area_b + area_g - inter
    l_iou = 1.0 - inter / jnp.maximum(union, 1e-9)

    cls_loss = jnp.sum(counts * l_cls, axis=(0, 1), keepdims=True) / n
    reg_loss = jnp.sum(counts * l_reg, axis=(0, 1), keepdims=True) / (4.0 * n)
    iou_loss = jnp.sum(counts * l_iou, axis=(0, 1), keepdims=True) / n
    total = cls_loss + reg_loss + iou_loss
    out_ref[...] = jnp.concatenate([cls_loss, reg_loss, iou_loss, total], axis=1)


def _combine(hist, clst, lab, bbt, glt, n):
    return pl.pallas_call(
        functools.partial(_combine_body, n=float(n)),
        out_shape=jax.ShapeDtypeStruct((1, 4), jnp.float32),
    )(hist, clst, lab, bbt, glt)


def kernel(cls_logits, bbox_pred, gt_labels, gt_boxes):
    n = bbox_pred.shape[0]
    f32 = jnp.float32
    cls_logits = cls_logits.astype(f32)
    bbox_pred = bbox_pred.astype(f32)
    gt_boxes = gt_boxes.astype(f32)
    gt_labels = gt_labels.astype(jnp.int32)

    bbt_all = jnp.zeros((4, _NP), f32).at[:, :n].set(bbox_pred.T)
    gtp = jnp.zeros((_GP, 4), f32).at[:_G].set(gt_boxes)
    matched = _match(bbt_all, gtp, n)

    hist = _hist_sc(matched.reshape(-1))

    cls_cat = jnp.concatenate([cls_logits[:_G], cls_logits[n - 1:n]], axis=0)
    bb_cat = jnp.concatenate([bbox_pred[:_G], bbox_pred[n - 1:n]], axis=0)
    gl_cat = jnp.concatenate([gt_boxes, gt_boxes[_G - 1:_G]], axis=0)
    lab_cat = jnp.concatenate([gt_labels, gt_labels[_G - 1:_G]], axis=0)

    clst = jnp.zeros((_C, _TBL), f32).at[:, :_G + 1].set(cls_cat.T)
    bbt = jnp.zeros((4, _TBL), f32).at[:, :_G + 1].set(bb_cat.T)
    glt = jnp.zeros((4, _TBL), f32).at[:, :_G + 1].set(gl_cat.T)
    lab = jnp.zeros((1, _TBL), jnp.int32).at[0, :_G + 1].set(lab_cat)

    out = _combine(hist, clst, lab, bbt, glt, n)
    return out.reshape(4)


# speedup vs baseline: 1.1692x; 1.1692x over previous
import functools

import jax
import jax.numpy as jnp
from jax import lax
from jax.experimental import pallas as pl
from jax.experimental.pallas import tpu as pltpu
from jax.experimental.pallas import tpu_sc as plsc

_C = 81
_G = 400
_GP = 512
_BINS = 416
_PAD_BIN = 415
_TBL = 408

_NW = 32
_CHUNK = 1568
_NP = _NW * _CHUNK

_BA = 1792


def _match_body(bbt_ref, gtp_ref, out_ref, *, n):
    b = bbt_ref[...]
    g = gtp_ref[...]
    ax1, ay1, ax2, ay2 = (b[i:i + 1, :] for i in range(4))
    gx1, gy1, gx2, gy2 = (g[:, i:i + 1] for i in range(4))

    area_a = (ax2 - ax1) * (ay2 - ay1)
    area_g = (gx2 - gx1) * (gy2 - gy1)
    w = jnp.maximum(jnp.minimum(ax2, gx2) - jnp.maximum(ax1, gx1), 0.0)
    h = jnp.maximum(jnp.minimum(ay2, gy2) - jnp.maximum(ay1, gy1), 0.0)
    inter = w * h
    r = inter / (area_a + area_g)

    bits = lax.bitcast_convert_type(r, jnp.int32)
    revj = (_GP - 1) - lax.broadcasted_iota(jnp.int32, (_G, 1), 0)
    key = jnp.bitwise_or(jnp.bitwise_and(bits, -_GP), revj)
    kmax = jnp.max(key, axis=0, keepdims=True)

    idx = (_GP - 1) - jnp.bitwise_and(kmax, _GP - 1)
    r_q = lax.bitcast_convert_type(jnp.bitwise_and(kmax, -_GP), jnp.float32)
    matched = jnp.where(r_q >= jnp.float32(1.0 / 3.0), idx, _G)
    aid = pl.program_id(0) * _BA + lax.broadcasted_iota(jnp.int32, (1, _BA), 1)
    out_ref[...] = jnp.where(aid < n, matched, _PAD_BIN)


def _match(bbt, gtp, n):
    return pl.pallas_call(
        functools.partial(_match_body, n=n),
        grid=(_NP // _BA,),
        in_specs=[
            pl.BlockSpec((4, _BA), lambda i: (0, i)),
            pl.BlockSpec((_G, 4), lambda i: (0, 0)),
        ],
        out_specs=pl.BlockSpec((1, _BA), lambda i: (0, i)),
        out_shape=jax.ShapeDtypeStruct((1, _NP), jnp.int32),
    )(bbt, gtp)


def _hist_body(idx_hbm, out_hbm, idx_v, hist_v, row_v):
    wid = lax.axis_index("s") * 2 + lax.axis_index("c")
    base = wid * _CHUNK
    pltpu.sync_copy(idx_hbm.at[pl.ds(base, _CHUNK)], idx_v)

    zeros = jnp.zeros((16,), jnp.float32)
    ones = jnp.ones((16,), jnp.float32)
    lane_off = lax.iota(jnp.int32, 16) * _BINS

    def _zero(i, _):
        for k in range(16):
            hist_v[pl.ds((i * 16 + k) * 16, 16)] = zeros
        return 0
    lax.fori_loop(0, _BINS // 16, _zero, 0)

    def _accum(i, _):
        for k in range(7):
            v = idx_v[pl.ds((i * 7 + k) * 16, 16)]
            plsc.addupdate_scatter(hist_v, [lane_off + v], ones)
        return 0
    lax.fori_loop(0, _CHUNK // 16 // 7, _accum, 0)

    def _reduce(c, _):
        acc = hist_v[pl.ds(c * 16, 16)]
        for r in range(1, 16):
            acc = acc + hist_v[pl.ds(c * 16 + r * _BINS, 16)]
        row_v[pl.ds(c * 16, 16)] = acc
        return 0
    lax.fori_loop(0, _BINS // 16, _reduce, 0)

    pltpu.sync_copy(row_v, out_hbm.at[wid])


def _hist_sc(flat_idx):
    return pl.kernel(
        _hist_body,
        out_type=jax.ShapeDtypeStruct((_NW, _BINS), jnp.float32),
        mesh=plsc.VectorSubcoreMesh(
            core_axis_name="c", subcore_axis_name="s",
            num_cores=2, num_subcores=16),
        scratch_types=[
            pltpu.VMEM((_CHUNK,), jnp.int32),
            pltpu.VMEM((16 * _BINS,), jnp.float32),
            pltpu.VMEM((_BINS,), jnp.float32),
        ],
        compiler_params=pltpu.CompilerParams(needs_layout_passes=False),
    )(flat_idx)


def _combine_body(hist_ref, clst_ref, lab_ref, bbt_ref, glt_ref, out_ref, n):
    counts = jnp.sum(hist_ref[...], axis=0, keepdims=True)[:, :_TBL]
    bin_id = lax.broadcasted_iota(jnp.int32, (1, _TBL), 1)
    counts = jnp.where(bin_id <= _G, counts, 0.0)

    logits = clst_ref[...]
    m = jnp.max(logits, axis=0, keepdims=True)
    sh = logits - m
    lse = jnp.log(jnp.sum(jnp.exp(sh), axis=0, keepdims=True))
    row = lax.broadcasted_iota(jnp.int32, (_C, _TBL), 0)
    sel = jnp.sum(jnp.where(row == lab_ref[...], sh, 0.0), axis=0, keepdims=True)
    l_cls = lse - sel

    b = bbt_ref[...]
    g = glt_ref[...]
    d = jnp.abs(b - g)
    sl1 = jnp.where(d < 1.0, 0.5 * d * d, d - 0.5)
    l_reg = jnp.sum(sl1, axis=0, keepdims=True)

    area_b = (b[2:3] - b[0:1]) * (b[3:4] - b[1:2])
    area_g = (g[2:3] - g[0:1]) * (g[3:4] - g[1:2])
    w = jnp.maximum(jnp.minimum(b[2:3], g[2:3]) - jnp.maximum(b[0:1], g[0:1]), 0.0)
    h = jnp.maximum(jnp.minimum(b[3:4], g[3:4]) - jnp.maximum(b[1:2], g[1:2]), 0.0)
    inter = w * h
    union = area_b + area_g - inter
    l_iou = 1.0 - inter / jnp.maximum(union, 1e-9)

    cls_loss = jnp.sum(counts * l_cls, axis=(0, 1), keepdims=True) / n
    reg_loss = jnp.sum(counts * l_reg, axis=(0, 1), keepdims=True) / (4.0 * n)
    iou_loss = jnp.sum(counts * l_iou, axis=(0, 1), keepdims=True) / n
    total = cls_loss + reg_loss + iou_loss
    out_ref[...] = jnp.concatenate([cls_loss, reg_loss, iou_loss, total], axis=1)


def _combine(hist, clst, lab, bbt, glt, n):
    return pl.pallas_call(
        functools.partial(_combine_body, n=float(n)),
        out_shape=jax.ShapeDtypeStruct((1, 4), jnp.float32),
    )(hist, clst, lab, bbt, glt)


def kernel(cls_logits, bbox_pred, gt_labels, gt_boxes):
    n = bbox_pred.shape[0]
    f32 = jnp.float32
    cls_logits = cls_logits.astype(f32)
    bbox_pred = bbox_pred.astype(f32)
    gt_boxes = gt_boxes.astype(f32)
    gt_labels = gt_labels.astype(jnp.int32)

    bbt_all = jnp.zeros((4, _NP), f32).at[:, :n].set(bbox_pred.T)
    gtp = gt_boxes
    matched = _match(bbt_all, gtp, n)

    hist = _hist_sc(matched.reshape(-1))

    cls_cat = jnp.concatenate([cls_logits[:_G], cls_logits[n - 1:n]], axis=0)
    bb_cat = jnp.concatenate([bbox_pred[:_G], bbox_pred[n - 1:n]], axis=0)
    gl_cat = jnp.concatenate([gt_boxes, gt_boxes[_G - 1:_G]], axis=0)
    lab_cat = jnp.concatenate([gt_labels, gt_labels[_G - 1:_G]], axis=0)

    clst = jnp.zeros((_C, _TBL), f32).at[:, :_G + 1].set(cls_cat.T)
    bbt = jnp.zeros((4, _TBL), f32).at[:, :_G + 1].set(bb_cat.T)
    glt = jnp.zeros((4, _TBL), f32).at[:, :_G + 1].set(gl_cat.T)
    lab = jnp.zeros((1, _TBL), jnp.int32).at[0, :_G + 1].set(lab_cat)

    out = _combine(hist, clst, lab, bbt, glt, n)
    return out.reshape(4)


# speedup vs baseline: 1.1718x; 1.0022x over previous
import functools

import jax
import jax.numpy as jnp
from jax import lax
from jax.experimental import pallas as pl
from jax.experimental.pallas import tpu as pltpu
from jax.experimental.pallas import tpu_sc as plsc

_C = 81
_G = 400
_GP = 512
_BINS = 416
_PAD_BIN = 415
_TBL = 408

_NW = 32
_CHUNK = 1568
_NP = _NW * _CHUNK

_BA = 1792


def _match_body(bbt_ref, gtp_ref, out_ref, *, n):
    b = bbt_ref[...]
    g = gtp_ref[...]
    ax1, ay1, ax2, ay2 = (b[i:i + 1, :] for i in range(4))
    gx1, gy1, gx2, gy2 = (g[:, i:i + 1] for i in range(4))

    area_a = (ax2 - ax1) * (ay2 - ay1)
    area_g = (gx2 - gx1) * (gy2 - gy1)
    w = jnp.maximum(jnp.minimum(ax2, gx2) - jnp.maximum(ax1, gx1), 0.0)
    h = jnp.maximum(jnp.minimum(ay2, gy2) - jnp.maximum(ay1, gy1), 0.0)
    inter = w * h
    r = inter / (area_a + area_g)

    bits = lax.bitcast_convert_type(r, jnp.int32)
    revj = (_GP - 1) - lax.broadcasted_iota(jnp.int32, (_G, 1), 0)
    key = jnp.bitwise_or(jnp.bitwise_and(bits, -_GP), revj)
    kmax = jnp.max(key, axis=0, keepdims=True)

    idx = (_GP - 1) - jnp.bitwise_and(kmax, _GP - 1)
    r_q = lax.bitcast_convert_type(jnp.bitwise_and(kmax, -_GP), jnp.float32)
    matched = jnp.where(r_q >= jnp.float32(1.0 / 3.0), idx, _G)
    aid = pl.program_id(0) * _BA + lax.broadcasted_iota(jnp.int32, (1, _BA), 1)
    out_ref[...] = jnp.where(aid < n, matched, _PAD_BIN)


def _match(bbt, gtp, n):
    return pl.pallas_call(
        functools.partial(_match_body, n=n),
        grid=(_NP // _BA,),
        in_specs=[
            pl.BlockSpec((4, _BA), lambda i: (0, i)),
            pl.BlockSpec((_G, 4), lambda i: (0, 0)),
        ],
        out_specs=pl.BlockSpec((1, _BA), lambda i: (0, i)),
        out_shape=jax.ShapeDtypeStruct((1, _NP), jnp.int32),
    )(bbt, gtp)


def _hist_body(idx_hbm, out_hbm, idx_v, hist_v, row_v):
    wid = lax.axis_index("s") * 2 + lax.axis_index("c")
    base = wid * _CHUNK
    pltpu.sync_copy(idx_hbm.at[pl.ds(base, _CHUNK)], idx_v)

    zeros = jnp.zeros((16,), jnp.float32)
    ones = jnp.ones((16,), jnp.float32)
    lane_off = lax.iota(jnp.int32, 16) * _BINS

    def _zero(i, _):
        for k in range(16):
            hist_v[pl.ds((i * 16 + k) * 16, 16)] = zeros
        return 0
    lax.fori_loop(0, _BINS // 16, _zero, 0)

    def _accum(i, _):
        for k in range(7):
            v = idx_v[pl.ds((i * 7 + k) * 16, 16)]
            plsc.addupdate_scatter(hist_v, [lane_off + v], ones)
        return 0
    lax.fori_loop(0, _CHUNK // 16 // 7, _accum, 0)

    def _reduce(c, _):
        acc = hist_v[pl.ds(c * 16, 16)]
        for r in range(1, 16):
            acc = acc + hist_v[pl.ds(c * 16 + r * _BINS, 16)]
        row_v[pl.ds(c * 16, 16)] = acc
        return 0
    lax.fori_loop(0, _BINS // 16, _reduce, 0)

    pltpu.sync_copy(row_v, out_hbm.at[wid])


def _hist_sc(flat_idx):
    return pl.kernel(
        _hist_body,
        out_type=jax.ShapeDtypeStruct((_NW, _BINS), jnp.float32),
        mesh=plsc.VectorSubcoreMesh(
            core_axis_name="c", subcore_axis_name="s",
            num_cores=2, num_subcores=16),
        scratch_types=[
            pltpu.VMEM((_CHUNK,), jnp.int32),
            pltpu.VMEM((16 * _BINS,), jnp.float32),
            pltpu.VMEM((_BINS,), jnp.float32),
        ],
        compiler_params=pltpu.CompilerParams(needs_layout_passes=False, disable_bounds_checks=True, disable_semaphore_checks=True),
    )(flat_idx)


def _combine_body(hist_ref, clst_ref, lab_ref, bbt_ref, glt_ref, out_ref, n):
    counts = jnp.sum(hist_ref[...], axis=0, keepdims=True)[:, :_TBL]
    bin_id = lax.broadcasted_iota(jnp.int32, (1, _TBL), 1)
    counts = jnp.where(bin_id <= _G, counts, 0.0)

    logits = clst_ref[...]
    m = jnp.max(logits, axis=0, keepdims=True)
    sh = logits - m
    lse = jnp.log(jnp.sum(jnp.exp(sh), axis=0, keepdims=True))
    row = lax.broadcasted_iota(jnp.int32, (_C, _TBL), 0)
    sel = jnp.sum(jnp.where(row == lab_ref[...], sh, 0.0), axis=0, keepdims=True)
    l_cls = lse - sel

    b = bbt_ref[...]
    g = glt_ref[...]
    d = jnp.abs(b - g)
    sl1 = jnp.where(d < 1.0, 0.5 * d * d, d - 0.5)
    l_reg = jnp.sum(sl1, axis=0, keepdims=True)

    area_b = (b[2:3] - b[0:1]) * (b[3:4] - b[1:2])
    area_g = (g[2:3] - g[0:1]) * (g[3:4] - g[1:2])
    w = jnp.maximum(jnp.minimum(b[2:3], g[2:3]) - jnp.maximum(b[0:1], g[0:1]), 0.0)
    h = jnp.maximum(jnp.minimum(b[3:4], g[3:4]) - jnp.maximum(b[1:2], g[1:2]), 0.0)
    inter = w * h
    union = area_b + area_g - inter
    l_iou = 1.0 - inter / jnp.maximum(union, 1e-9)

    cls_loss = jnp.sum(counts * l_cls, axis=(0, 1), keepdims=True) / n
    reg_loss = jnp.sum(counts * l_reg, axis=(0, 1), keepdims=True) / (4.0 * n)
    iou_loss = jnp.sum(counts * l_iou, axis=(0, 1), keepdims=True) / n
    total = cls_loss + reg_loss + iou_loss
    out_ref[...] = jnp.concatenate([cls_loss, reg_loss, iou_loss, total], axis=1)


def _combine(hist, clst, lab, bbt, glt, n):
    return pl.pallas_call(
        functools.partial(_combine_body, n=float(n)),
        out_shape=jax.ShapeDtypeStruct((1, 4), jnp.float32),
    )(hist, clst, lab, bbt, glt)


def kernel(cls_logits, bbox_pred, gt_labels, gt_boxes):
    n = bbox_pred.shape[0]
    f32 = jnp.float32
    cls_logits = cls_logits.astype(f32)
    bbox_pred = bbox_pred.astype(f32)
    gt_boxes = gt_boxes.astype(f32)
    gt_labels = gt_labels.astype(jnp.int32)

    bbt_all = jnp.zeros((4, _NP), f32).at[:, :n].set(bbox_pred.T)
    gtp = gt_boxes
    matched = _match(bbt_all, gtp, n)

    hist = _hist_sc(matched.reshape(-1))

    cls_cat = jnp.concatenate([cls_logits[:_G], cls_logits[n - 1:n]], axis=0)
    bb_cat = jnp.concatenate([bbox_pred[:_G], bbox_pred[n - 1:n]], axis=0)
    gl_cat = jnp.concatenate([gt_boxes, gt_boxes[_G - 1:_G]], axis=0)
    lab_cat = jnp.concatenate([gt_labels, gt_labels[_G - 1:_G]], axis=0)

    clst = jnp.zeros((_C, _TBL), f32).at[:, :_G + 1].set(cls_cat.T)
    bbt = jnp.zeros((4, _TBL), f32).at[:, :_G + 1].set(bb_cat.T)
    glt = jnp.zeros((4, _TBL), f32).at[:, :_G + 1].set(gl_cat.T)
    lab = jnp.zeros((1, _TBL), jnp.int32).at[0, :_G + 1].set(lab_cat)

    out = _combine(hist, clst, lab, bbt, glt, n)
    return out.reshape(4)
